# baseline (speedup 1.0000x reference)
import functools

import jax
import jax.numpy as jnp
from jax import lax
from jax.experimental import pallas as pl
from jax.experimental.pallas import tpu as pltpu

_EPS = 1e-5
_VMEM_LIMIT = 64 * 1024 * 1024


def _fused_block_kernel(x_ref, bw_ref, bb_ref, wd_ref, b2_ref, w3_ref, b3_ref,
                        o_ref, *, half, mid, H, W, pad):
    L = H * W
    xb = x_ref[0].reshape(x_ref.shape[1], L)
    y = jnp.dot(bw_ref[...], xb, preferred_element_type=jnp.float32)
    y = y + bb_ref[...]
    o_ref[0, :half] = y[:half].reshape(half, H, W)
    h1 = jnp.maximum(y[half:], 0.0)

    zp = jnp.zeros((mid, pad), jnp.float32)
    hp = jnp.concatenate([zp, h1, zp], axis=1)
    wcol = lax.broadcasted_iota(jnp.int32, (1, L), 1) % W
    mask_l = (wcol != 0).astype(jnp.float32)
    mask_r = (wcol != W - 1).astype(jnp.float32)
    acc = jnp.zeros((mid, L), jnp.float32)
    for dh in (-1, 0, 1):
        for dw in (-1, 0, 1):
            t = 3 * (dh + 1) + (dw + 1)
            s = pad + dh * W + dw
            tap = hp[:, s:s + L]
            if dw == -1:
                tap = tap * mask_l
            elif dw == 1:
                tap = tap * mask_r
            acc = acc + tap * wd_ref[:, t:t + 1]
    h2 = acc + b2_ref[...]

    out = jnp.dot(w3_ref[...], h2, preferred_element_type=jnp.float32)
    out = jnp.maximum(out + b3_ref[...], 0.0)
    o_ref[0, half:] = out.reshape(out.shape[0], H, W)


def _bn_fold(gamma, beta, mean, var):
    s = gamma * lax.rsqrt(var + _EPS)
    return s, beta - mean * s


def kernel(x, channel_choice, bn1_beta, bn1_gamma, bn1_mean, bn1_var,
           bn2_beta, bn2_gamma, bn2_mean, bn2_var,
           bn3_beta, bn3_gamma, bn3_mean, bn3_var,
           w1, w3, wd):
    B, C, H, W = x.shape
    mid = w1.shape[0]
    outputs = w3.shape[0]
    L = H * W

    s1, b1 = _bn_fold(bn1_gamma, bn1_beta, bn1_mean, bn1_var)
    s2, b2 = _bn_fold(bn2_gamma, bn2_beta, bn2_mean, bn2_var)
    s3, b3 = _bn_fold(bn3_gamma, bn3_beta, bn3_mean, bn3_var)

    mask = channel_choice[0, :mid]
    w1_eff = w1 * (mask * s1)[:, None]

    half = C // 2
    sel = jnp.zeros((half, C), jnp.float32).at[
        jnp.arange(half), 2 * jnp.arange(half)].set(1.0)
    w1_big = jnp.zeros((mid, C), jnp.float32).at[:, 1::2].set(w1_eff)
    big_w = jnp.concatenate([sel, w1_big], axis=0)
    big_b = jnp.concatenate([jnp.zeros((half,), jnp.float32), b1])[:, None]

    wd_t = (wd * s2[None, :]).T
    w3_eff = w3 * s3[:, None]

    pad = 32
    kern = functools.partial(_fused_block_kernel, half=half, mid=mid, H=H,
                             W=W, pad=pad)
    out = pl.pallas_call(
        kern,
        out_shape=jax.ShapeDtypeStruct((B, half + outputs, H, W), jnp.float32),
        grid_spec=pltpu.PrefetchScalarGridSpec(
            num_scalar_prefetch=0,
            grid=(B,),
            in_specs=[
                pl.BlockSpec((1, C, H, W), lambda b: (b, 0, 0, 0)),
                pl.BlockSpec((half + mid, C), lambda b: (0, 0)),
                pl.BlockSpec((half + mid, 1), lambda b: (0, 0)),
                pl.BlockSpec((mid, 9), lambda b: (0, 0)),
                pl.BlockSpec((mid, 1), lambda b: (0, 0)),
                pl.BlockSpec((outputs, mid), lambda b: (0, 0)),
                pl.BlockSpec((outputs, 1), lambda b: (0, 0)),
            ],
            out_specs=pl.BlockSpec((1, half + outputs, H, W),
                                   lambda b: (b, 0, 0, 0)),
        ),
        compiler_params=pltpu.CompilerParams(
            dimension_semantics=("parallel",),
            vmem_limit_bytes=_VMEM_LIMIT,
        ),
    )(x, big_w, big_b, wd_t, b2[:, None], w3_eff, b3[:, None])
    return out


# speedup vs baseline: 8.6117x; 8.6117x over previous
import functools

import jax
import jax.numpy as jnp
from jax import lax
from jax.experimental import pallas as pl
from jax.experimental.pallas import tpu as pltpu

_EPS = 1e-5
_VMEM_LIMIT = 100 * 1024 * 1024


def _fused_block_kernel(x_ref, bw_ref, pr_ref, wd_ref, w3_ref, o_ref, *,
                        half, mid, H, W, bblk):
    L = H * W
    R = L * bblk
    C = x_ref.shape[2]
    pad = 29 * bblk
    xb = x_ref[...].reshape(R, C)
    y = jnp.dot(xb, bw_ref[...], preferred_element_type=jnp.float32)
    y = y + pr_ref[0:1, :]
    o_left = y[:, :half]
    h1 = jnp.maximum(y[:, half:], 0.0)

    zp = jnp.zeros((pad, mid), jnp.float32)
    hp = jnp.concatenate([zp, h1, zp], axis=0)
    wco = (lax.broadcasted_iota(jnp.int32, (R, 1), 0) // bblk) % W
    mask_l = (wco != 0).astype(jnp.float32)
    mask_r = (wco != W - 1).astype(jnp.float32)
    acc = jnp.zeros((R, mid), jnp.float32)
    for dh in (-1, 0, 1):
        for dw in (-1, 0, 1):
            t = 3 * (dh + 1) + (dw + 1)
            s = pad + (dh * W + dw) * bblk
            tap = hp[s:s + R]
            if dw == -1:
                tap = tap * mask_l
            elif dw == 1:
                tap = tap * mask_r
            acc = acc + tap * wd_ref[t:t + 1, :]
    h2 = acc + pr_ref[1:2, :mid]

    out = jnp.dot(h2, w3_ref[...], preferred_element_type=jnp.float32)
    out = jnp.maximum(out + pr_ref[2:3, half:], 0.0)
    o_ref[...] = jnp.concatenate([o_left, out], axis=1).reshape(L, bblk, C)


def _bn_fold(gamma, beta, mean, var):
    s = gamma * lax.rsqrt(var + _EPS)
    return s, beta - mean * s


def kernel(x, channel_choice, bn1_beta, bn1_gamma, bn1_mean, bn1_var,
           bn2_beta, bn2_gamma, bn2_mean, bn2_var,
           bn3_beta, bn3_gamma, bn3_mean, bn3_var,
           w1, w3, wd):
    B, C, H, W = x.shape
    mid = w1.shape[0]
    outputs = w3.shape[0]
    half = C // 2
    L = H * W

    s1, b1 = _bn_fold(bn1_gamma, bn1_beta, bn1_mean, bn1_var)
    s2, b2 = _bn_fold(bn2_gamma, bn2_beta, bn2_mean, bn2_var)
    s3, b3 = _bn_fold(bn3_gamma, bn3_beta, bn3_mean, bn3_var)

    mask = channel_choice[0, :mid]
    w1_eff = w1 * (mask * s1)[:, None]

    sel = jnp.zeros((half, C), jnp.float32).at[
        jnp.arange(half), 2 * jnp.arange(half)].set(1.0)
    w1_big = jnp.zeros((mid, C), jnp.float32).at[:, 1::2].set(w1_eff)
    big_w = jnp.concatenate([sel, w1_big], axis=0).T

    pr = jnp.zeros((8, C), jnp.float32)
    pr = pr.at[0, half:].set(b1)
    pr = pr.at[1, :mid].set(b2)
    pr = pr.at[2, half:].set(b3)

    wd16 = jnp.zeros((16, mid), jnp.float32).at[:9].set(wd * s2[None, :])
    w3_t = (w3 * s3[:, None]).T

    bblk = 8
    xt = x.transpose(2, 3, 0, 1).reshape(L, B, C)
    kern = functools.partial(_fused_block_kernel, half=half, mid=mid, H=H,
                             W=W, bblk=bblk)
    out = pl.pallas_call(
        kern,
        out_shape=jax.ShapeDtypeStruct((L, B, half + outputs), jnp.float32),
        grid_spec=pltpu.PrefetchScalarGridSpec(
            num_scalar_prefetch=0,
            grid=(B // bblk,),
            in_specs=[
                pl.BlockSpec((L, bblk, C), lambda b: (0, b, 0)),
                pl.BlockSpec((C, half + mid), lambda b: (0, 0)),
                pl.BlockSpec((8, C), lambda b: (0, 0)),
                pl.BlockSpec((16, mid), lambda b: (0, 0)),
                pl.BlockSpec((mid, outputs), lambda b: (0, 0)),
            ],
            out_specs=pl.BlockSpec((L, bblk, half + outputs),
                                   lambda b: (0, b, 0)),
        ),
        compiler_params=pltpu.CompilerParams(
            dimension_semantics=("parallel",),
            vmem_limit_bytes=_VMEM_LIMIT,
        ),
    )(xt, big_w, pr, wd16, w3_t)
    return out.reshape(H, W, B, half + outputs).transpose(2, 3, 0, 1)


# speedup vs baseline: 13.3721x; 1.5528x over previous
import functools

import jax
import jax.numpy as jnp
from jax import lax
from jax.experimental import pallas as pl
from jax.experimental.pallas import tpu as pltpu

_EPS = 1e-5
_VMEM_LIMIT = 100 * 1024 * 1024


def _fused_block_kernel(x_ref, bw_ref, pr_ref, wd_ref, w3_ref, o_ref, *,
                        half, mid, H, W, bblk):
    L = H * W
    C = x_ref.shape[2]
    dn = (((2,), (0,)), ((), ()))
    y = lax.dot_general(x_ref[...], bw_ref[...], dn,
                        preferred_element_type=jnp.float32)
    y = y + pr_ref[0:1, :].reshape(1, 1, C)
    o_left = y[:, :, :half]
    h1 = jnp.maximum(y[:, :, half:], 0.0)

    zp = jnp.zeros((29, bblk, mid), jnp.float32)
    hp = jnp.concatenate([zp, h1, zp], axis=0)
    wco = lax.broadcasted_iota(jnp.int32, (L, 1, 1), 0) % W
    mask_l = (wco != 0).astype(jnp.float32)
    mask_r = (wco != W - 1).astype(jnp.float32)
    acc = jnp.zeros((L, bblk, mid), jnp.float32)
    for dh in (-1, 0, 1):
        for dw in (-1, 0, 1):
            t = 3 * (dh + 1) + (dw + 1)
            s = 29 + dh * W + dw
            tap = hp[s:s + L]
            if dw == -1:
                tap = tap * mask_l
            elif dw == 1:
                tap = tap * mask_r
            acc = acc + tap * wd_ref[t:t + 1, :].reshape(1, 1, mid)
    h2 = acc + pr_ref[1:2, :mid].reshape(1, 1, mid)

    out = lax.dot_general(h2, w3_ref[...], dn,
                          preferred_element_type=jnp.float32)
    out = jnp.maximum(out + pr_ref[2:3, half:].reshape(1, 1, C - half), 0.0)
    o_ref[...] = jnp.concatenate([o_left, out], axis=2)


def _bn_fold(gamma, beta, mean, var):
    s = gamma * lax.rsqrt(var + _EPS)
    return s, beta - mean * s


def kernel(x, channel_choice, bn1_beta, bn1_gamma, bn1_mean, bn1_var,
           bn2_beta, bn2_gamma, bn2_mean, bn2_var,
           bn3_beta, bn3_gamma, bn3_mean, bn3_var,
           w1, w3, wd):
    B, C, H, W = x.shape
    mid = w1.shape[0]
    outputs = w3.shape[0]
    half = C // 2
    L = H * W

    s1, b1 = _bn_fold(bn1_gamma, bn1_beta, bn1_mean, bn1_var)
    s2, b2 = _bn_fold(bn2_gamma, bn2_beta, bn2_mean, bn2_var)
    s3, b3 = _bn_fold(bn3_gamma, bn3_beta, bn3_mean, bn3_var)

    mask = channel_choice[0, :mid]
    w1_eff = w1 * (mask * s1)[:, None]

    left = jnp.stack([jnp.eye(half, dtype=jnp.float32),
                      jnp.zeros((half, half), jnp.float32)],
                     axis=1).reshape(C, half)
    right = jnp.stack([jnp.zeros((half, mid), jnp.float32), w1_eff.T],
                      axis=1).reshape(C, mid)
    big_w = jnp.concatenate([left, right], axis=1)

    zh = jnp.zeros((half,), jnp.float32)
    pr = jnp.stack([jnp.concatenate([zh, b1]),
                    jnp.concatenate([b2, jnp.zeros((C - mid,), jnp.float32)]),
                    jnp.concatenate([zh, b3])])
    pr = jnp.concatenate([pr, jnp.zeros((5, C), jnp.float32)], axis=0)

    wd16 = jnp.concatenate([wd * s2[None, :], jnp.zeros((7, mid),
                                                        jnp.float32)], axis=0)
    w3_t = (w3 * s3[:, None]).T

    bblk = 8
    xt = x.transpose(2, 3, 0, 1).reshape(L, B, C)
    kern = functools.partial(_fused_block_kernel, half=half, mid=mid, H=H,
                             W=W, bblk=bblk)
    out = pl.pallas_call(
        kern,
        out_shape=jax.ShapeDtypeStruct((L, B, half + outputs), jnp.float32),
        grid_spec=pltpu.PrefetchScalarGridSpec(
            num_scalar_prefetch=0,
            grid=(B // bblk,),
            in_specs=[
                pl.BlockSpec((L, bblk, C), lambda b: (0, b, 0)),
                pl.BlockSpec((C, half + mid), lambda b: (0, 0)),
                pl.BlockSpec((8, C), lambda b: (0, 0)),
                pl.BlockSpec((16, mid), lambda b: (0, 0)),
                pl.BlockSpec((mid, outputs), lambda b: (0, 0)),
            ],
            out_specs=pl.BlockSpec((L, bblk, half + outputs),
                                   lambda b: (0, b, 0)),
        ),
        compiler_params=pltpu.CompilerParams(
            dimension_semantics=("parallel",),
            vmem_limit_bytes=_VMEM_LIMIT,
        ),
    )(xt, big_w, pr, wd16, w3_t)
    return out.reshape(H, W, B, half + outputs).transpose(2, 3, 0, 1)


# speedup vs baseline: 13.9949x; 1.0466x over previous
import functools

import jax
import jax.numpy as jnp
from jax import lax
from jax.experimental import pallas as pl
from jax.experimental.pallas import tpu as pltpu

_EPS = 1e-5
_VMEM_LIMIT = 100 * 1024 * 1024


def _fused_block_kernel(x_ref, pk_ref, w1_ref, w3_ref, o_ref, *,
                        half, mid, H, W, bblk):
    L = H * W
    C = x_ref.shape[2]
    f32 = jnp.float32

    def bn_fold(i):
        beta, gamma, mean, var = (pk_ref[i + j:i + j + 1, :] for j in range(4))
        s = gamma * lax.rsqrt(var + _EPS)
        return s, beta - mean * s

    s1, b1 = bn_fold(0)
    s2, b2 = bn_fold(4)
    s3, b3 = bn_fold(8)
    cc = pk_ref[12:13, :]

    row = lax.broadcasted_iota(jnp.int32, (C, half), 0)
    col = lax.broadcasted_iota(jnp.int32, (C, half), 1)
    left = (row == 2 * col).astype(f32)
    odd_sel = (row == 2 * col + 1).astype(f32)
    right = lax.dot_general(odd_sel, w1_ref[...], (((1,), (1,)), ((), ())),
                            preferred_element_type=f32)
    right = right * (cc * s1)
    big_w = jnp.concatenate([left, right], axis=1)

    dn = (((2,), (0,)), ((), ()))
    y = lax.dot_general(x_ref[...], big_w, dn, preferred_element_type=f32)
    h1 = jnp.maximum(y[:, :, half:] + b1.reshape(1, 1, mid), 0.0)
    o_left = y[:, :, :half]

    wde = pk_ref[16:25, :] * s2
    zp = jnp.zeros((29, bblk, mid), f32)
    hp = jnp.concatenate([zp, h1, zp], axis=0)
    wco = lax.broadcasted_iota(jnp.int32, (L, 1, 1), 0) % W
    mask_l = (wco != 0).astype(f32)
    mask_r = (wco != W - 1).astype(f32)
    acc = jnp.zeros((L, bblk, mid), f32)
    for dh in (-1, 0, 1):
        for dw in (-1, 0, 1):
            t = 3 * (dh + 1) + (dw + 1)
            tap = hp[29 + dh * W + dw:29 + dh * W + dw + L]
            if dw == -1:
                tap = tap * mask_l
            elif dw == 1:
                tap = tap * mask_r
            acc = acc + tap * wde[t:t + 1, :].reshape(1, 1, mid)
    h2 = acc + b2.reshape(1, 1, mid)

    out = lax.dot_general(h2, w3_ref[...], (((2,), (1,)), ((), ())),
                          preferred_element_type=f32)
    out = out * s3.reshape(1, 1, C - half) + b3.reshape(1, 1, C - half)
    out = jnp.maximum(out, 0.0)
    o_ref[...] = jnp.concatenate([o_left, out], axis=2)


def kernel(x, channel_choice, bn1_beta, bn1_gamma, bn1_mean, bn1_var,
           bn2_beta, bn2_gamma, bn2_mean, bn2_var,
           bn3_beta, bn3_gamma, bn3_mean, bn3_var,
           w1, w3, wd):
    B, C, H, W = x.shape
    mid = w1.shape[0]
    outputs = w3.shape[0]
    half = C // 2
    L = H * W

    pk = jnp.stack([bn1_beta, bn1_gamma, bn1_mean, bn1_var,
                    bn2_beta, bn2_gamma, bn2_mean, bn2_var,
                    bn3_beta, bn3_gamma, bn3_mean, bn3_var,
                    channel_choice[0, :mid],
                    jnp.zeros((mid,), jnp.float32),
                    jnp.zeros((mid,), jnp.float32),
                    jnp.zeros((mid,), jnp.float32)])
    pk = jnp.concatenate([pk, wd, jnp.zeros((32 - 16 - wd.shape[0], mid),
                                            jnp.float32)], axis=0)

    bblk = 8
    xt = x.transpose(2, 3, 0, 1).reshape(L, B, C)
    kern = functools.partial(_fused_block_kernel, half=half, mid=mid, H=H,
                             W=W, bblk=bblk)
    out = pl.pallas_call(
        kern,
        out_shape=jax.ShapeDtypeStruct((L, B, half + outputs), jnp.float32),
        grid_spec=pltpu.PrefetchScalarGridSpec(
            num_scalar_prefetch=0,
            grid=(B // bblk,),
            in_specs=[
                pl.BlockSpec((L, bblk, C), lambda b: (0, b, 0)),
                pl.BlockSpec((32, mid), lambda b: (0, 0)),
                pl.BlockSpec((mid, half), lambda b: (0, 0)),
                pl.BlockSpec((outputs, mid), lambda b: (0, 0)),
            ],
            out_specs=pl.BlockSpec((L, bblk, half + outputs),
                                   lambda b: (0, b, 0)),
        ),
        compiler_params=pltpu.CompilerParams(
            dimension_semantics=("parallel",),
            vmem_limit_bytes=_VMEM_LIMIT,
        ),
    )(xt, pk, w1, w3)
    return out.reshape(H, W, B, half + outputs).transpose(2, 3, 0, 1)
